# Initial kernel scaffold; baseline (speedup 1.0000x reference)
import jax
import jax.numpy as jnp
from jax.experimental import pallas as pl


def kernel(node_feats, spatial_feats, cond_feats, W_node, b_node, W_s, b_s, W_cond, b_cond, W_lin, node_i_ids, node_j_ids, batch_ids):
    raise NotImplementedError("write your pallas kernel here")


# speedup vs baseline: 6.6066x; 6.6066x over previous
import functools

import jax
import jax.numpy as jnp
from jax import lax
from jax.experimental import pallas as pl
from jax.experimental.pallas import tpu as pltpu
from jax.experimental.pallas import tpu_sc as plsc

_NC = 2
_NS = 16
_NW = _NC * _NS


def _proj_body(x_ref, wnT_ref, bn_ref, cond_ref, wcT_ref, bc_ref,
               nf_ref, gb_ref):
    nf = jnp.dot(x_ref[...], wnT_ref[...], preferred_element_type=jnp.float32)
    nf_ref[...] = jnp.maximum(nf + bn_ref[...], 0.0)
    gb = jnp.dot(cond_ref[...], wcT_ref[...],
                 preferred_element_type=jnp.float32)
    gb_ref[...] = gb + bc_ref[...]


def _node_proj(node_feats, W_node_T, b_node, cond_feats, W_cond_T, b_cond):
    n, d = node_feats.shape
    b, twod = cond_feats.shape[0], W_cond_T.shape[1]
    return pl.pallas_call(
        _proj_body,
        out_shape=(
            jax.ShapeDtypeStruct((n, W_node_T.shape[1]), jnp.float32),
            jax.ShapeDtypeStruct((b, twod), jnp.float32),
        ),
    )(node_feats, W_node_T, b_node, cond_feats, W_cond_T, b_cond)


def _gather_mul(nf, node_i_ids, node_j_ids, batch_ids, chunk):
    e = node_i_ids.shape[0]
    n, d = nf.shape
    epw = e // _NW
    nchunk = epw // chunk
    groups = d // 16

    mesh = plsc.VectorSubcoreMesh(core_axis_name="c", subcore_axis_name="s")

    @functools.partial(
        pl.kernel,
        mesh=mesh,
        out_type=(
            jax.ShapeDtypeStruct((e, d), jnp.float32),
            jax.ShapeDtypeStruct((e,), jnp.int32),
        ),
        scratch_types=[
            pltpu.VMEM((chunk,), jnp.int32),
            pltpu.VMEM((chunk,), jnp.int32),
            pltpu.VMEM((chunk, d), jnp.float32),
            pltpu.VMEM((chunk, d), jnp.float32),
            pltpu.VMEM((chunk,), jnp.int32),
            pltpu.SemaphoreType.DMA,
            pltpu.SemaphoreType.DMA,
            pltpu.SemaphoreType.DMA,
        ],
    )
    def k(nf_hbm, ii_hbm, jj_hbm, bid_hbm, joint_hbm, ebid_hbm,
          idx_i_v, idx_j_v, rows_i_v, rows_j_v, eb_v, sem_i, sem_j, sem_b):
        wid = lax.axis_index("s") * _NC + lax.axis_index("c")
        base = wid * epw

        def chunk_body(kk, carry):
            off = base + kk * chunk
            pltpu.sync_copy(ii_hbm.at[pl.ds(off, chunk)], idx_i_v)
            pltpu.sync_copy(jj_hbm.at[pl.ds(off, chunk)], idx_j_v)
            cp_i = pltpu.async_copy(nf_hbm.at[idx_i_v], rows_i_v, sem_i)
            cp_j = pltpu.async_copy(nf_hbm.at[idx_j_v], rows_j_v, sem_j)
            cp_b = pltpu.async_copy(bid_hbm.at[idx_j_v], eb_v, sem_b)
            cp_i.wait()
            cp_j.wait()
            cp_b.wait()

            def mul_row(r, c2):
                for g in range(groups):
                    sl = pl.ds(g * 16, 16)
                    rows_i_v[r, sl] = rows_i_v[r, sl] * rows_j_v[r, sl]
                return c2

            lax.fori_loop(0, chunk, mul_row, 0, unroll=False)

            pltpu.sync_copy(rows_i_v, joint_hbm.at[pl.ds(off, chunk)])
            pltpu.sync_copy(eb_v, ebid_hbm.at[pl.ds(off, chunk)])
            return carry

        lax.fori_loop(0, nchunk, chunk_body, 0, unroll=False)

    return k(nf, node_i_ids, node_j_ids, batch_ids)


def _film_body(joint_ref, sp_ref, eb_ref, gb_ref, wsT_ref, bs_ref,
               wljT_ref, wlsT_ref, out_ref, *, be, nb, dd):
    s = jnp.dot(sp_ref[...], wsT_ref[...], preferred_element_type=jnp.float32)
    s = jnp.maximum(s + bs_ref[...], 0.0)
    x = (jnp.dot(joint_ref[...], wljT_ref[...],
                 preferred_element_type=jnp.float32)
         + jnp.dot(s, wlsT_ref[...], preferred_element_type=jnp.float32))
    mu = jnp.mean(x, axis=1, keepdims=True)
    xc = x - mu
    var = jnp.mean(xc * xc, axis=1, keepdims=True)
    xn = xc * lax.rsqrt(var + 1e-5)
    oh = (eb_ref[...] == lax.broadcasted_iota(jnp.int32, (be, nb), 1))
    gbv = jnp.dot(oh.astype(jnp.float32), gb_ref[...],
                  preferred_element_type=jnp.float32)
    gamma = gbv[:, :dd] + 1.0
    beta = gbv[:, dd:]
    out_ref[...] = jnp.maximum(xn * gamma + beta, 0.0)


def _film(joint, spatial, ebid_col, gb, wsT, bs, wljT, wlsT, be):
    e, d = joint.shape
    nb = gb.shape[0]
    sp_dim = spatial.shape[1]
    sh = wsT.shape[1]
    grid = (e // be,)
    return pl.pallas_call(
        functools.partial(_film_body, be=be, nb=nb, dd=d),
        grid=grid,
        in_specs=[
            pl.BlockSpec((be, d), lambda i: (i, 0)),
            pl.BlockSpec((be, sp_dim), lambda i: (i, 0)),
            pl.BlockSpec((be, 1), lambda i: (i, 0)),
            pl.BlockSpec((nb, 2 * d), lambda i: (0, 0)),
            pl.BlockSpec((sp_dim, sh), lambda i: (0, 0)),
            pl.BlockSpec((1, sh), lambda i: (0, 0)),
            pl.BlockSpec((d, d), lambda i: (0, 0)),
            pl.BlockSpec((sh, d), lambda i: (0, 0)),
        ],
        out_specs=pl.BlockSpec((be, d), lambda i: (i, 0)),
        out_shape=jax.ShapeDtypeStruct((e, d), jnp.float32),
    )(joint, spatial, ebid_col, gb, wsT, bs, wljT, wlsT)


def kernel(node_feats, spatial_feats, cond_feats, W_node, b_node, W_s, b_s,
           W_cond, b_cond, W_lin, node_i_ids, node_j_ids, batch_ids):
    n, nd = node_feats.shape
    e = node_i_ids.shape[0]
    d = W_node.shape[0]
    sp_dim = spatial_feats.shape[1]
    ns = W_s.shape[0]
    sh = 32

    nf, gb = _node_proj(node_feats, W_node.T, b_node[None, :],
                        cond_feats, W_cond.T, b_cond[None, :])

    joint, ebid = _gather_mul(nf, node_i_ids, node_j_ids, batch_ids, chunk=80)

    wsT = jnp.zeros((sp_dim, sh), jnp.float32).at[:, :ns].set(W_s.T)
    bs = jnp.zeros((1, sh), jnp.float32).at[:, :ns].set(b_s[None, :])
    wljT = W_lin[:, :d].T
    wlsT = jnp.zeros((sh, d), jnp.float32).at[:ns, :].set(W_lin[:, d:].T)

    return _film(joint, spatial_feats, ebid.reshape(e, 1), gb,
                 wsT, bs, wljT, wlsT, be=2000)


# speedup vs baseline: 6.8182x; 1.0320x over previous
import functools

import jax
import jax.numpy as jnp
from jax import lax
from jax.experimental import pallas as pl
from jax.experimental.pallas import tpu as pltpu
from jax.experimental.pallas import tpu_sc as plsc

_NC = 2
_NS = 16
_NW = _NC * _NS


def _proj_body(x_ref, wnT_ref, bn_ref, cond_ref, wcT_ref, bc_ref,
               nf_ref, gb_ref):
    nf = jnp.dot(x_ref[...], wnT_ref[...], preferred_element_type=jnp.float32)
    nf_ref[...] = jnp.maximum(nf + bn_ref[...], 0.0)
    gb = jnp.dot(cond_ref[...], wcT_ref[...],
                 preferred_element_type=jnp.float32)
    gb_ref[...] = gb + bc_ref[...]


def _node_proj(node_feats, W_node_T, b_node, cond_feats, W_cond_T, b_cond):
    n, d = node_feats.shape
    b, twod = cond_feats.shape[0], W_cond_T.shape[1]
    return pl.pallas_call(
        _proj_body,
        out_shape=(
            jax.ShapeDtypeStruct((n, W_node_T.shape[1]), jnp.float32),
            jax.ShapeDtypeStruct((b, twod), jnp.float32),
        ),
    )(node_feats, W_node_T, b_node, cond_feats, W_cond_T, b_cond)


def _gather_mul(nf, node_i_ids, node_j_ids, batch_ids, chunk):
    e = node_i_ids.shape[0]
    n, d = nf.shape
    epw = e // _NW
    nchunk = epw // chunk
    groups = d // 16

    ii2 = node_i_ids.reshape(_NW, nchunk, chunk)
    jj2 = node_j_ids.reshape(_NW, nchunk, chunk)

    mesh = plsc.VectorSubcoreMesh(core_axis_name="c", subcore_axis_name="s")

    @functools.partial(
        pl.kernel,
        mesh=mesh,
        out_type=(
            jax.ShapeDtypeStruct((e, d), jnp.float32),
            jax.ShapeDtypeStruct((_NW, nchunk, chunk), jnp.int32),
        ),
        scratch_types=[
            pltpu.VMEM((nchunk, chunk), jnp.int32),
            pltpu.VMEM((nchunk, chunk), jnp.int32),
            pltpu.VMEM((nchunk, chunk), jnp.int32),
            pltpu.VMEM((2, chunk, d), jnp.float32),
            pltpu.VMEM((2, chunk, d), jnp.float32),
            pltpu.VMEM((2, chunk, d), jnp.float32),
            pltpu.SemaphoreType.DMA,
            pltpu.SemaphoreType.DMA,
            pltpu.SemaphoreType.DMA,
            pltpu.SemaphoreType.DMA,
            pltpu.SemaphoreType.DMA,
            pltpu.SemaphoreType.DMA,
            pltpu.SemaphoreType.DMA,
            pltpu.SemaphoreType.DMA,
        ],
    )
    def k(nf_hbm, ii_hbm, jj_hbm, bid_hbm, joint_hbm, ebid_hbm,
          idx_i_v, idx_j_v, eb_v, rows_i_v, rows_j_v, joint_v,
          sgi0, sgi1, sgj0, sgj1, sgb0, sgb1, so0, so1):
        sgi = (sgi0, sgi1)
        sgj = (sgj0, sgj1)
        sgb = (sgb0, sgb1)
        so = (so0, so1)
        wid = lax.axis_index("s") * _NC + lax.axis_index("c")
        base = wid * epw

        pltpu.sync_copy(ii_hbm.at[wid], idx_i_v)
        pltpu.sync_copy(jj_hbm.at[wid], idx_j_v)

        def issue(kk, b):
            pltpu.make_async_copy(
                nf_hbm.at[idx_i_v.at[kk]], rows_i_v.at[b], sgi[b]).start()
            pltpu.make_async_copy(
                nf_hbm.at[idx_j_v.at[kk]], rows_j_v.at[b], sgj[b]).start()
            pltpu.make_async_copy(
                bid_hbm.at[idx_j_v.at[kk]], eb_v.at[kk], sgb[b]).start()

        def wait_gather(kk, b):
            pltpu.make_async_copy(
                nf_hbm.at[idx_i_v.at[kk]], rows_i_v.at[b], sgi[b]).wait()
            pltpu.make_async_copy(
                nf_hbm.at[idx_j_v.at[kk]], rows_j_v.at[b], sgj[b]).wait()
            pltpu.make_async_copy(
                bid_hbm.at[idx_j_v.at[kk]], eb_v.at[kk], sgb[b]).wait()

        def wait_out(b):
            pltpu.make_async_copy(
                joint_v.at[b], joint_hbm.at[pl.ds(base, chunk)], so[b]).wait()

        def mul(b):
            def mrow(r, c2):
                for g in range(groups):
                    sl = pl.ds(g * 16, 16)
                    joint_v[b, r, sl] = rows_i_v[b, r, sl] * rows_j_v[b, r, sl]
                return c2

            lax.fori_loop(0, chunk, mrow, 0, unroll=4)

        def issue_out(kk, b):
            pltpu.make_async_copy(
                joint_v.at[b],
                joint_hbm.at[pl.ds(base + kk * chunk, chunk)], so[b]).start()

        def step(kk, b, skip_wait_out, issue_next):
            if not skip_wait_out:
                wait_out(b)
            wait_gather(kk, b)
            if issue_next:
                issue(kk + 1, 1 - b)
            mul(b)
            issue_out(kk, b)

        issue(0, 0)
        step(0, 0, skip_wait_out=True, issue_next=True)
        step(1, 1, skip_wait_out=True, issue_next=True)

        def pair(t, c2):
            kk = 2 + 2 * t
            step(kk, 0, skip_wait_out=False, issue_next=True)
            step(kk + 1, 1, skip_wait_out=False, issue_next=True)
            return c2

        lax.fori_loop(0, (nchunk - 3) // 2, pair, 0, unroll=False)
        step(nchunk - 1, (nchunk - 1) % 2,
             skip_wait_out=False, issue_next=False)
        wait_out((nchunk - 1) % 2)
        wait_out(nchunk % 2)
        pltpu.sync_copy(eb_v, ebid_hbm.at[wid])

    joint, ebid = k(nf, ii2, jj2, batch_ids)
    return joint, ebid.reshape(e)


def _film_body(joint_ref, sp_ref, eb_ref, gb_ref, wsT_ref, bs_ref,
               wljT_ref, wlsT_ref, out_ref, *, be, nb, dd):
    s = jnp.dot(sp_ref[...], wsT_ref[...], preferred_element_type=jnp.float32)
    s = jnp.maximum(s + bs_ref[...], 0.0)
    x = (jnp.dot(joint_ref[...], wljT_ref[...],
                 preferred_element_type=jnp.float32)
         + jnp.dot(s, wlsT_ref[...], preferred_element_type=jnp.float32))
    mu = jnp.mean(x, axis=1, keepdims=True)
    xc = x - mu
    var = jnp.mean(xc * xc, axis=1, keepdims=True)
    xn = xc * lax.rsqrt(var + 1e-5)
    oh = (eb_ref[...] == lax.broadcasted_iota(jnp.int32, (be, nb), 1))
    gbv = jnp.dot(oh.astype(jnp.float32), gb_ref[...],
                  preferred_element_type=jnp.float32)
    gamma = gbv[:, :dd] + 1.0
    beta = gbv[:, dd:]
    out_ref[...] = jnp.maximum(xn * gamma + beta, 0.0)


def _film(joint, spatial, ebid_col, gb, wsT, bs, wljT, wlsT, be):
    e, d = joint.shape
    nb = gb.shape[0]
    sp_dim = spatial.shape[1]
    sh = wsT.shape[1]
    grid = (e // be,)
    return pl.pallas_call(
        functools.partial(_film_body, be=be, nb=nb, dd=d),
        grid=grid,
        in_specs=[
            pl.BlockSpec((be, d), lambda i: (i, 0)),
            pl.BlockSpec((be, sp_dim), lambda i: (i, 0)),
            pl.BlockSpec((be, 1), lambda i: (i, 0)),
            pl.BlockSpec((nb, 2 * d), lambda i: (0, 0)),
            pl.BlockSpec((sp_dim, sh), lambda i: (0, 0)),
            pl.BlockSpec((1, sh), lambda i: (0, 0)),
            pl.BlockSpec((d, d), lambda i: (0, 0)),
            pl.BlockSpec((sh, d), lambda i: (0, 0)),
        ],
        out_specs=pl.BlockSpec((be, d), lambda i: (i, 0)),
        out_shape=jax.ShapeDtypeStruct((e, d), jnp.float32),
    )(joint, spatial, ebid_col, gb, wsT, bs, wljT, wlsT)


def kernel(node_feats, spatial_feats, cond_feats, W_node, b_node, W_s, b_s,
           W_cond, b_cond, W_lin, node_i_ids, node_j_ids, batch_ids):
    n, nd = node_feats.shape
    e = node_i_ids.shape[0]
    d = W_node.shape[0]
    sp_dim = spatial_feats.shape[1]
    ns = W_s.shape[0]
    sh = 32

    nf, gb = _node_proj(node_feats, W_node.T, b_node[None, :],
                        cond_feats, W_cond.T, b_cond[None, :])

    joint, ebid = _gather_mul(nf, node_i_ids, node_j_ids, batch_ids, chunk=80)

    wsT = jnp.zeros((sp_dim, sh), jnp.float32).at[:, :ns].set(W_s.T)
    bs = jnp.zeros((1, sh), jnp.float32).at[:, :ns].set(b_s[None, :])
    wljT = W_lin[:, :d].T
    wlsT = jnp.zeros((sh, d), jnp.float32).at[:ns, :].set(W_lin[:, d:].T)

    return _film(joint, spatial_feats, ebid.reshape(e, 1), gb,
                 wsT, bs, wljT, wlsT, be=4000)
